# Initial kernel scaffold; baseline (speedup 1.0000x reference)
import jax
import jax.numpy as jnp
from jax.experimental import pallas as pl


def kernel(users, items_pos, items_neg, embed_user_w, embed_item_w, embed_user_visual_w, imgfeat_item_visual, trans_e, bias_visual):
    raise NotImplementedError("write your pallas kernel here")


# speedup vs baseline: 2.1716x; 2.1716x over previous
import functools

import jax
import jax.numpy as jnp
from jax import lax
from jax.experimental import pallas as pl
from jax.experimental.pallas import tpu as pltpu
from jax.experimental.pallas import tpu_sc as plsc

N_USERS = 100000
N_ITEMS = 100000
D_LAT = 128
D_IMG = 4096
B = 16384
REG = 1e-05

_NC = 2
_NS = 16
_NW = _NC * _NS
_BPW = B // _NW
_SCHUNK = 128
_GCHUNK = 16


def _sc_gather_all(users, items_pos, items_neg, uw, iw, uvw, img):
    mesh = plsc.VectorSubcoreMesh(core_axis_name="c", subcore_axis_name="s")

    @functools.partial(
        pl.kernel,
        mesh=mesh,
        out_type=[
            jax.ShapeDtypeStruct((B, D_LAT), jnp.float32),
            jax.ShapeDtypeStruct((B, D_LAT), jnp.float32),
            jax.ShapeDtypeStruct((B, D_LAT), jnp.float32),
            jax.ShapeDtypeStruct((B, D_LAT), jnp.float32),
            jax.ShapeDtypeStruct((B, D_IMG), jnp.float32),
            jax.ShapeDtypeStruct((B, D_IMG), jnp.float32),
        ],
        scratch_types=[
            pltpu.VMEM((_SCHUNK,), jnp.int32),
            pltpu.VMEM((_GCHUNK,), jnp.int32),
            pltpu.VMEM((_SCHUNK, D_LAT), jnp.float32),
            pltpu.VMEM((_GCHUNK, D_IMG), jnp.float32),
            pltpu.SemaphoreType.DMA,
        ],
    )
    def gather_kernel(users_h, ipos_h, ineg_h, uw_h, iw_h, uvw_h, img_h,
                      ulat_o, ilp_o, iln_o, uvis_o, gp_o, gn_o,
                      idx_s, idx_g, sbuf, gbuf, sem):
        wid = lax.axis_index("s") * _NC + lax.axis_index("c")
        base = wid * _BPW

        def small_body(c, carry):
            off = base + c * _SCHUNK
            pltpu.sync_copy(users_h.at[pl.ds(off, _SCHUNK)], idx_s)
            pltpu.async_copy(uw_h.at[idx_s], sbuf, sem).wait()
            pltpu.sync_copy(sbuf, ulat_o.at[pl.ds(off, _SCHUNK)])
            pltpu.async_copy(uvw_h.at[idx_s], sbuf, sem).wait()
            pltpu.sync_copy(sbuf, uvis_o.at[pl.ds(off, _SCHUNK)])
            pltpu.sync_copy(ipos_h.at[pl.ds(off, _SCHUNK)], idx_s)
            pltpu.async_copy(iw_h.at[idx_s], sbuf, sem).wait()
            pltpu.sync_copy(sbuf, ilp_o.at[pl.ds(off, _SCHUNK)])
            pltpu.sync_copy(ineg_h.at[pl.ds(off, _SCHUNK)], idx_s)
            pltpu.async_copy(iw_h.at[idx_s], sbuf, sem).wait()
            pltpu.sync_copy(sbuf, iln_o.at[pl.ds(off, _SCHUNK)])
            return carry

        lax.fori_loop(0, _BPW // _SCHUNK, small_body, 0)

        def big_body(c, carry):
            off = base + c * _GCHUNK
            pltpu.sync_copy(ipos_h.at[pl.ds(off, _GCHUNK)], idx_g)
            pltpu.async_copy(img_h.at[idx_g], gbuf, sem).wait()
            pltpu.sync_copy(gbuf, gp_o.at[pl.ds(off, _GCHUNK)])
            pltpu.sync_copy(ineg_h.at[pl.ds(off, _GCHUNK)], idx_g)
            pltpu.async_copy(img_h.at[idx_g], gbuf, sem).wait()
            pltpu.sync_copy(gbuf, gn_o.at[pl.ds(off, _GCHUNK)])
            return carry

        lax.fori_loop(0, _BPW // _GCHUNK, big_body, 0)

    return gather_kernel(users, items_pos, items_neg, uw, iw, uvw, img)


_R = 256


def _tc_loss_kernel(ulat, ilp, iln, uvis, gp, gn, te, bias,
                    base_o, reg_o):
    step = pl.program_id(0)
    te_v = te[...]
    bias_v = bias[...]
    u_vis = uvis[...]
    w = jnp.dot(u_vis, te_v, preferred_element_type=jnp.float32) + bias_v

    u_lat = ulat[...]
    sp = (jnp.sum(w * gp[...], axis=1, keepdims=True)
          + jnp.sum(u_lat * ilp[...], axis=1, keepdims=True))
    sn = (jnp.sum(w * gn[...], axis=1, keepdims=True)
          + jnp.sum(u_lat * iln[...], axis=1, keepdims=True))
    nd = sn - sp
    sploss = jnp.maximum(nd, 0.0) + jnp.log(1.0 + jnp.exp(-jnp.abs(nd)))
    base_part = jnp.sum(sploss) * (1.0 / B)
    reg_part = (0.5 * REG) * (
        jnp.sum(u_lat * u_lat) + jnp.sum(ilp[...] * ilp[...])
        + jnp.sum(iln[...] * iln[...]) + jnp.sum(u_vis * u_vis))

    @pl.when(step == 0)
    def _init():
        base_o[0, 0] = 0.0
        reg_o[0, 0] = (0.5 * REG) * (jnp.sum(bias_v * bias_v)
                                     + jnp.sum(te_v * te_v))

    base_o[0, 0] += base_part
    reg_o[0, 0] += reg_part


def kernel(users, items_pos, items_neg, embed_user_w, embed_item_w,
           embed_user_visual_w, imgfeat_item_visual, trans_e, bias_visual):
    users = users.astype(jnp.int32)
    items_pos = items_pos.astype(jnp.int32)
    items_neg = items_neg.astype(jnp.int32)

    ulat, ilp, iln, uvis, gp, gn = _sc_gather_all(
        users, items_pos, items_neg, embed_user_w, embed_item_w,
        embed_user_visual_w, imgfeat_item_visual)

    bias_row = bias_visual.reshape(1, D_IMG)
    grid = (B // _R,)
    row_blk = lambda i: (i, 0)
    rep_blk = lambda i: (0, 0)
    base_o, reg_o = pl.pallas_call(
        _tc_loss_kernel,
        grid=grid,
        in_specs=[
            pl.BlockSpec((_R, D_LAT), row_blk),
            pl.BlockSpec((_R, D_LAT), row_blk),
            pl.BlockSpec((_R, D_LAT), row_blk),
            pl.BlockSpec((_R, D_LAT), row_blk),
            pl.BlockSpec((_R, D_IMG), row_blk),
            pl.BlockSpec((_R, D_IMG), row_blk),
            pl.BlockSpec((D_LAT, D_IMG), rep_blk),
            pl.BlockSpec((1, D_IMG), rep_blk),
        ],
        out_specs=[
            pl.BlockSpec((1, 1), rep_blk, memory_space=pltpu.SMEM),
            pl.BlockSpec((1, 1), rep_blk, memory_space=pltpu.SMEM),
        ],
        out_shape=[
            jax.ShapeDtypeStruct((1, 1), jnp.float32),
            jax.ShapeDtypeStruct((1, 1), jnp.float32),
        ],
    )(ulat, ilp, iln, uvis, gp, gn, trans_e, bias_row)

    return (base_o[0, 0], reg_o[0, 0])
